# baseline (speedup 1.0000x reference)
import math

import jax
import jax.numpy as jnp
from jax.experimental import pallas as pl

N = 32768
B = 16
DIM_FEAT = 512
DIM_TGT = 2
P = 8
GATING = 100.0
BASE = 2.0

BN = 4096
NB = N // BN

_LN_G = math.log(GATING)


def _fused_kernel(x_ref, ann_ref, bi_ref, proto_ref, w_ref, out_ref, src_ref):
    i = pl.program_id(0)
    x = x_ref[...]
    p = proto_ref[...]
    pn = p / jnp.maximum(
        jnp.sqrt(jnp.sum(p * p, axis=1, keepdims=True)), 1e-8)
    inv = 1.0 / jnp.maximum(
        jnp.sqrt(jnp.sum(x * x, axis=1, keepdims=True)), 1e-8)
    sim = jnp.dot(x, pn.T, preferred_element_type=jnp.float32,
                  precision=jax.lax.Precision.HIGHEST) * inv
    deltas = jnp.exp((sim - 1.0) * _LN_G)
    not_deltas = jnp.exp((-sim - 1.0) * _LN_G)
    and_d = jnp.min(deltas, axis=1, keepdims=True)
    or_d = jnp.max(deltas, axis=1, keepdims=True)
    w = w_ref[...]
    src = (
        jnp.dot(deltas, w[:, :P].T, preferred_element_type=jnp.float32,
                  precision=jax.lax.Precision.HIGHEST)
        + jnp.dot(not_deltas, w[:, P:2 * P].T,
                  preferred_element_type=jnp.float32,
                  precision=jax.lax.Precision.HIGHEST)
        + and_d * w[:, 2 * P][None, :]
        + or_d * w[:, 2 * P + 1][None, :]
    )
    ann = ann_ref[...].astype(jnp.float32)
    src = src * (1.0 + ann)
    src_ref[...] = src
    bi = bi_ref[...].reshape(1, BN)
    seg = jax.lax.broadcasted_iota(jnp.int32, (B, BN), 0)
    onehot = (bi == seg).astype(jnp.float32)
    part = jnp.dot(onehot, src, preferred_element_type=jnp.float32,
                   precision=jax.lax.Precision.HIGHEST)

    @pl.when(i == 0)
    def _init():
        out_ref[...] = part

    @pl.when(i > 0)
    def _acc():
        out_ref[...] += part


def kernel(x, annotations, mask, batch_idx, prototypes, W):
    del mask
    ann = annotations
    bi = batch_idx.reshape(NB, 1, BN)
    out, src = pl.pallas_call(
        _fused_kernel,
        grid=(NB,),
        in_specs=[
            pl.BlockSpec((BN, DIM_FEAT), lambda i: (i, 0)),
            pl.BlockSpec((BN, 1), lambda i: (i, 0)),
            pl.BlockSpec((1, 1, BN), lambda i: (i, 0, 0)),
            pl.BlockSpec((P, DIM_FEAT), lambda i: (0, 0)),
            pl.BlockSpec((DIM_TGT, 2 * P + 2), lambda i: (0, 0)),
        ],
        out_specs=[
            pl.BlockSpec((B, DIM_TGT), lambda i: (0, 0)),
            pl.BlockSpec((BN, DIM_TGT), lambda i: (i, 0)),
        ],
        out_shape=[
            jax.ShapeDtypeStruct((B, DIM_TGT), jnp.float32),
            jax.ShapeDtypeStruct((N, DIM_TGT), jnp.float32),
        ],
    )(x, ann, bi, prototypes, W)
    return (out, src)


# speedup vs baseline: 2.3863x; 2.3863x over previous
import math

import jax
import jax.numpy as jnp
from jax.experimental import pallas as pl

N = 32768
B = 16
DIM_FEAT = 512
DIM_TGT = 2
P = 8
GATING = 100.0
BASE = 2.0

BN = 4096
NB = N // BN

_LN_G = math.log(GATING)


def _fused_kernel(x_ref, ann_ref, bi_ref, proto_ref, w_ref, out_ref, src_ref):
    i = pl.program_id(0)
    x = x_ref[...]
    p = proto_ref[...]
    pn = p / jnp.maximum(
        jnp.sqrt(jnp.sum(p * p, axis=1, keepdims=True)), 1e-8)
    xn = x * (1.0 / jnp.maximum(
        jnp.sqrt(jnp.sum(x * x, axis=1, keepdims=True)), 1e-8))
    sim = jnp.dot(xn, pn.T, preferred_element_type=jnp.float32)
    deltas = jnp.exp((sim - 1.0) * _LN_G)
    not_deltas = jnp.exp((-sim - 1.0) * _LN_G)
    and_d = jnp.min(deltas, axis=1, keepdims=True)
    or_d = jnp.max(deltas, axis=1, keepdims=True)
    w = w_ref[...]
    feats = jnp.concatenate([deltas, not_deltas, and_d, or_d], axis=1)
    src = jnp.dot(feats, w.T, preferred_element_type=jnp.float32)
    ann = ann_ref[...].astype(jnp.float32)
    src = src * (1.0 + ann)
    src_ref[...] = src
    bi = bi_ref[...].reshape(1, BN)
    seg = jax.lax.broadcasted_iota(jnp.int32, (B, BN), 0)
    onehot = (bi == seg).astype(jnp.float32)
    src_hi = src.astype(jnp.bfloat16).astype(jnp.float32)
    src_lo = src - src_hi
    part = (
        jnp.dot(onehot, src_hi, preferred_element_type=jnp.float32)
        + jnp.dot(onehot, src_lo, preferred_element_type=jnp.float32)
    )

    @pl.when(i == 0)
    def _init():
        out_ref[...] = part

    @pl.when(i > 0)
    def _acc():
        out_ref[...] += part


def kernel(x, annotations, mask, batch_idx, prototypes, W):
    del mask
    ann = annotations
    bi = batch_idx.reshape(NB, 1, BN)
    out, src = pl.pallas_call(
        _fused_kernel,
        grid=(NB,),
        in_specs=[
            pl.BlockSpec((BN, DIM_FEAT), lambda i: (i, 0)),
            pl.BlockSpec((BN, 1), lambda i: (i, 0)),
            pl.BlockSpec((1, 1, BN), lambda i: (i, 0, 0)),
            pl.BlockSpec((P, DIM_FEAT), lambda i: (0, 0)),
            pl.BlockSpec((DIM_TGT, 2 * P + 2), lambda i: (0, 0)),
        ],
        out_specs=[
            pl.BlockSpec((B, DIM_TGT), lambda i: (0, 0)),
            pl.BlockSpec((BN, DIM_TGT), lambda i: (i, 0)),
        ],
        out_shape=[
            jax.ShapeDtypeStruct((B, DIM_TGT), jnp.float32),
            jax.ShapeDtypeStruct((N, DIM_TGT), jnp.float32),
        ],
    )(x, ann, bi, prototypes, W)
    return (out, src)


# speedup vs baseline: 3.7542x; 1.5733x over previous
import functools
import math

import jax
import jax.numpy as jnp
from jax import lax
from jax.experimental import pallas as pl
from jax.experimental.pallas import tpu as pltpu
from jax.experimental.pallas import tpu_sc as plsc

N = 32768
B = 16
DIM_FEAT = 512
DIM_TGT = 2
P = 8
GATING = 100.0
BASE = 2.0

BN = 4096
NB = N // BN

_LN_G = math.log(GATING)

NC = 2
NS = 16
L = 16
NW = NC * NS
RW = N // NW
NBINS = B * L


def _tc_kernel(x_ref, ann_ref, proto_ref, w_ref, srct_ref):
    x = x_ref[...]
    p = proto_ref[...]
    pn = p / jnp.maximum(
        jnp.sqrt(jnp.sum(p * p, axis=1, keepdims=True)), 1e-8)
    xn = x / jnp.maximum(
        jnp.sqrt(jnp.sum(x * x, axis=1, keepdims=True)), 1e-8)
    sim = jnp.dot(xn, pn.T, preferred_element_type=jnp.float32)
    simt = sim.T
    deltas = jnp.exp((simt - 1.0) * _LN_G)
    not_deltas = jnp.exp((-simt - 1.0) * _LN_G)
    and_d = jnp.min(deltas, axis=0, keepdims=True)
    or_d = jnp.max(deltas, axis=0, keepdims=True)
    feats = jnp.concatenate([deltas, not_deltas, and_d, or_d], axis=0)
    srct = jnp.dot(w_ref[...], feats,
                   preferred_element_type=jnp.float32)
    ann = ann_ref[...].reshape(1, BN).astype(jnp.float32)
    srct_ref[...] = srct * (1.0 + ann)


def _sc_segsum(srct_hbm, bi_hbm, out_hbm, s0_v, s1_v, bi_v, acc_v, shared,
               big_v, outb_v):
    c = lax.axis_index("c")
    s = lax.axis_index("s")
    wid = s * NC + c
    base = wid * RW
    pltpu.sync_copy(srct_hbm.at[0, pl.ds(base, RW)], s0_v)
    pltpu.sync_copy(srct_hbm.at[1, pl.ds(base, RW)], s1_v)
    pltpu.sync_copy(bi_hbm.at[pl.ds(base, RW)], bi_v)
    zero16 = jnp.zeros((L,), jnp.float32)
    for j in range(2 * B):
        acc_v[pl.ds(j * L, L)] = zero16
    lanes = lax.iota(jnp.int32, L)

    def body(k, carry):
        o = k * L
        idx = lanes * B + bi_v[pl.ds(o, L)]
        plsc.addupdate_scatter(acc_v, [idx], s0_v[pl.ds(o, L)])
        plsc.addupdate_scatter(acc_v, [idx + NBINS], s1_v[pl.ds(o, L)])
        return carry

    lax.fori_loop(0, RW // L, body, 0)
    pltpu.sync_copy(acc_v, shared.at[s])
    plsc.subcore_barrier()

    @pl.when(s == 0)
    def _final():
        pltpu.sync_copy(shared, big_v)
        for t in range(DIM_TGT):
            tot = jnp.zeros((B,), jnp.float32)
            for j in range(NS):
                for l in range(L):
                    tot = tot + big_v[j, pl.ds(t * NBINS + l * B, B)]
            outb_v[t, pl.ds(0, B)] = tot
        pltpu.sync_copy(outb_v, out_hbm.at[c])


_sc_segsum_call = functools.partial(
    pl.kernel,
    out_type=jax.ShapeDtypeStruct((NC, DIM_TGT, B), jnp.float32),
    mesh=plsc.VectorSubcoreMesh(core_axis_name="c", subcore_axis_name="s"),
    compiler_params=pltpu.CompilerParams(needs_layout_passes=False),
    scratch_types=[
        pltpu.VMEM((RW,), jnp.float32),
        pltpu.VMEM((RW,), jnp.float32),
        pltpu.VMEM((RW,), jnp.int32),
        pltpu.VMEM((2 * NBINS,), jnp.float32),
        pltpu.VMEM_SHARED((NS, 2 * NBINS), jnp.float32),
        pltpu.VMEM((NS, 2 * NBINS), jnp.float32),
        pltpu.VMEM((DIM_TGT, B), jnp.float32),
    ],
)(_sc_segsum)


def kernel(x, annotations, mask, batch_idx, prototypes, W):
    del mask
    ann = annotations.reshape(NB, 1, BN)
    srct = pl.pallas_call(
        _tc_kernel,
        grid=(NB,),
        in_specs=[
            pl.BlockSpec((BN, DIM_FEAT), lambda i: (i, 0)),
            pl.BlockSpec((1, 1, BN), lambda i: (i, 0, 0)),
            pl.BlockSpec((P, DIM_FEAT), lambda i: (0, 0)),
            pl.BlockSpec((DIM_TGT, 2 * P + 2), lambda i: (0, 0)),
        ],
        out_specs=pl.BlockSpec((DIM_TGT, BN), lambda i: (0, i)),
        out_shape=jax.ShapeDtypeStruct((DIM_TGT, N), jnp.float32),
    )(x, ann, prototypes, W)
    out_parts = _sc_segsum_call(srct, batch_idx.astype(jnp.int32))
    return (jnp.sum(out_parts, axis=0).T, srct.T)
